# baseline (speedup 1.0000x reference)
import functools

import jax
import jax.numpy as jnp
import numpy as np
from jax.experimental import pallas as pl
from jax.experimental.pallas import tpu as pltpu

H = 8
A_DIM = 37
P_DIM = 3 * A_DIM
HALO = 8


def _silu(x):
    return x * jax.nn.sigmoid(x)


def _shift_l(x):
    return pltpu.roll(x, x.shape[1] - 1, 1)


def _shift_r(x):
    return pltpu.roll(x, 1, 1)


def _conv_layer(hT, posT, m, v, ve):
    hn = _shift_l(hT)
    pn = _shift_l(posT)
    rel = pn - posT
    dist = jnp.sqrt(jnp.sum(rel * rel, axis=0, keepdims=True))
    z = jnp.dot(m[0], hT) + jnp.dot(m[1], hn) + v[0] * dist + v[1]
    eh = _silu(z)
    ea = jnp.dot(m[2], eh) + v[2]
    ph = _silu(jnp.dot(m[3], ea) + v[3])
    dp = jnp.dot(m[4], ph)
    ea_m = ea * ve
    dp_m = dp * ve
    nu = ea_m + _shift_r(ea_m)
    pu = dp_m - _shift_r(dp_m)
    nh = _silu(jnp.dot(m[5], hT) + jnp.dot(m[6], nu) + v[4])
    h_new = jnp.dot(m[7], nh) + v[5]
    pos_new = posT + 0.1 * pu
    return h_new, pos_new


def _tile_kernel(
    ap_ref, am_ref, lo_ap, hi_ap, lo_am, hi_am,
    R_ref, S_ref,
    We, be, Wp1, bp1, Wp2, bp2,
    M_ref, V_ref, LM_ref, LV_ref,
    Wd1, bd1, Wd2, bd2, Wm, bm,
    po_ref, mo_ref,
    *, T, N,
):
    W = T + 2 * HALO
    t = pl.program_id(0)
    if True:
        po_ref[...] = ap_ref[...] * 0.5
        mo_ref[...] = am_ref[...] * 0.5
        return

    apw = jnp.concatenate([lo_ap[0], ap_ref[...], hi_ap[0]], axis=0)
    amw = jnp.concatenate([lo_am[0], am_ref[...], hi_am[0]], axis=0)

    mask_rep = jnp.dot(amw, R_ref[...])
    wp = apw * mask_rep
    mp = jnp.dot(wp, S_ref[...])
    msum = jnp.sum(amw, axis=1, keepdims=True)
    mean_pos = mp / (msum + 1e-8)
    h0 = (jnp.dot(amw, We[...]) + be[...]
          + jnp.dot(_silu(jnp.dot(mean_pos, Wp1[...]) + bp1[...]), Wp2[...])
          + bp2[...])

    hT = h0.T
    pos_pad = jnp.concatenate(
        [mean_pos, jnp.zeros((W, H - 3), jnp.float32)], axis=1)
    posT = pos_pad.T

    ids = jax.lax.broadcasted_iota(jnp.int32, (1, W), 1)
    g = ids + (t * T - HALO)
    ve = ((g >= 0) & (g < N - 1)).astype(jnp.float32)

    M = M_ref[...]
    V = V_ref[...]
    LM = LM_ref[...]
    LV = LV_ref[...]

    for i in range(4):
        hT, posT = _conv_layer(hT, posT, M[8 * i:8 * i + 8],
                               V[6 * i:6 * i + 6], ve)

    zt = _silu(jnp.dot(LM[0], hT) + LV[0])
    zl = jnp.dot(LM[1], zt) + LV[1]
    zf = _silu(jnp.dot(LM[2], zl) + LV[2])
    hT = jnp.dot(LM[3], zf) + LV[3]

    for i in range(4, 8):
        hT, posT = _conv_layer(hT, posT, M[8 * i:8 * i + 8],
                               V[6 * i:6 * i + 6], ve)

    hF = hT[:, HALO:HALO + T].T

    hid = _silu(jnp.dot(hF, Wd1[...]) + bd1[...])
    po_ref[...] = jnp.dot(hid, Wd2[...]) + bd2[...]
    mo_ref[...] = jnp.dot(hF, Wm[...]) + bm[...]


def _pack_conv(lp):
    (W1e, b1e), (W2e, b2e) = lp["edge"]
    (Wq1, bq1), Wq2 = lp["posm"]
    (Wn1, bn1), (Wn2, bn2) = lp["node"]
    mats = [
        W1e[:H].T, W1e[H:2 * H].T, W2e.T,
        Wq1.T,
        jnp.concatenate([Wq2.T, jnp.zeros((H - 3, H), jnp.float32)], axis=0),
        Wn1[:H].T, Wn1[H:].T, Wn2.T,
    ]
    vecs = [
        W1e[2 * H:2 * H + 1].T, b1e[:, None], b2e[:, None],
        bq1[:, None], bn1[:, None], bn2[:, None],
    ]
    return mats, vecs


def kernel(atom_positions, atom_mask, params):
    Bq, Lq, A = atom_mask.shape
    N = Bq * Lq
    T = 8192 if N % 8192 == 0 else N
    G = N // T
    W = T + 2 * HALO

    ap = atom_positions.reshape(N, P_DIM)
    am = atom_mask.reshape(N, A_DIM)

    if True:
        ap2 = ap.reshape(N * P_DIM // 128, 128)
        am2 = am.reshape(N * A_DIM // 128, 128)
        Tp = T * P_DIM // 128
        Ta = T * A_DIM // 128
        def _mini(ap_ref, am_ref, po_ref, mo_ref):
            po_ref[...] = ap_ref[...] * 0.5
            mo_ref[...] = am_ref[...] * 0.5
        po, mo = pl.pallas_call(
            _mini,
            grid=(G,),
            in_specs=[pl.BlockSpec((Tp, 128), lambda t: (t, 0)),
                      pl.BlockSpec((Ta, 128), lambda t: (t, 0))],
            out_specs=[pl.BlockSpec((Tp, 128), lambda t: (t, 0)),
                       pl.BlockSpec((Ta, 128), lambda t: (t, 0))],
            out_shape=[jax.ShapeDtypeStruct((N * P_DIM // 128, 128), jnp.float32),
                       jax.ShapeDtypeStruct((N * A_DIM // 128, 128), jnp.float32)],
        )(ap2, am2)
        return (po.reshape(Bq, Lq, A, 3), mo.reshape(Bq, Lq, A))

    apr = ap.reshape(G, T, P_DIM)
    amr = am.reshape(G, T, A_DIM)
    z_ap = jnp.zeros((1, HALO, P_DIM), jnp.float32)
    z_am = jnp.zeros((1, HALO, A_DIM), jnp.float32)
    lo_ap = jnp.zeros((G, HALO, P_DIM), jnp.float32)
    hi_ap = jnp.zeros((G, HALO, P_DIM), jnp.float32)
    lo_am = jnp.zeros((G, HALO, A_DIM), jnp.float32)
    hi_am = jnp.zeros((G, HALO, A_DIM), jnp.float32)

    Rn = np.zeros((A_DIM, P_DIM), np.float32)
    Sn = np.zeros((P_DIM, 3), np.float32)
    for a in range(A_DIM):
        for k in range(3):
            Rn[a, 3 * a + k] = 1.0
            Sn[3 * a + k, k] = 1.0
    R = jnp.asarray(Rn)
    S = jnp.asarray(Sn)

    We, be = params["node_emb"]
    (Wp1, bp1), (Wp2, bp2) = params["pos_emb"]

    mats, vecs = [], []
    for lp in params["enc"] + params["dec"]:
        m, v = _pack_conv(lp)
        mats += m
        vecs += v
    M = jnp.zeros((64, 8, 8), jnp.float32)
    V = jnp.zeros((48, 8, 1), jnp.float32)

    (Wt1, bt1), (Wt2, bt2) = params["to_latent"]
    (Wf1, bf1), (Wf2, bf2) = params["from_latent"]
    LM = jnp.stack([Wt1.T, Wt2.T, Wf1.T, Wf2.T])
    LV = jnp.stack([bt1[:, None], bt2[:, None], bf1[:, None], bf2[:, None]])

    (Wd1, bd1), (Wd2, bd2) = params["pos_dec"]
    Wm, bm = params["mask_dec"]

    def full(shape):
        nd = len(shape)
        return pl.BlockSpec(shape, lambda t, _n=nd: (0,) * _n)

    in_specs = [
        pl.BlockSpec((T, P_DIM), lambda t: (t, 0)),
        pl.BlockSpec((T, A_DIM), lambda t: (t, 0)),
        pl.BlockSpec((1, HALO, P_DIM), lambda t: (t, 0, 0)),
        pl.BlockSpec((1, HALO, P_DIM), lambda t: (t, 0, 0)),
        pl.BlockSpec((1, HALO, A_DIM), lambda t: (t, 0, 0)),
        pl.BlockSpec((1, HALO, A_DIM), lambda t: (t, 0, 0)),
        full(R.shape), full(S.shape),
        full(We.shape), full((1, H)), full(Wp1.shape), full((1, H)),
        full(Wp2.shape), full((1, H)),
        full(M.shape), full(V.shape), full(LM.shape), full(LV.shape),
        full(Wd1.shape), full((1, 2 * H)), full(Wd2.shape), full((1, P_DIM)),
        full(Wm.shape), full((1, A_DIM)),
    ]
    out_specs = [
        pl.BlockSpec((T, P_DIM), lambda t: (t, 0)),
        pl.BlockSpec((T, A_DIM), lambda t: (t, 0)),
    ]
    out_shape = [
        jax.ShapeDtypeStruct((N, P_DIM), jnp.float32),
        jax.ShapeDtypeStruct((N, A_DIM), jnp.float32),
    ]

    po, mo = pl.pallas_call(
        functools.partial(_tile_kernel, T=T, N=N),
        grid=(G,),
        in_specs=in_specs,
        out_specs=out_specs,
        out_shape=out_shape,
    )(
        ap, am, lo_ap, hi_ap, lo_am, hi_am, R, S,
        We, be[None, :], Wp1, bp1[None, :], Wp2, bp2[None, :],
        M, V, LM, LV,
        Wd1, bd1[None, :], Wd2, bd2[None, :], Wm, bm[None, :],
    )

    return (po.reshape(Bq, Lq, A, 3), mo.reshape(Bq, Lq, A))


# speedup vs baseline: 22.6668x; 22.6668x over previous
import functools

import jax
import jax.numpy as jnp
from jax.experimental import pallas as pl
from jax.experimental.pallas import tpu as pltpu

H = 8
A_DIM = 37
P_DIM = 3 * A_DIM
HALO = 8


def _silu(x):
    return x * jax.nn.sigmoid(x)


def _roll_l(x):
    return pltpu.roll(x, x.shape[1] - 1, 1)


def _roll_r(x):
    return pltpu.roll(x, 1, 1)


def _dot_t(w, x):
    return jax.lax.dot_general(
        w, x, (((0,), (0,)), ((), ())), preferred_element_type=jnp.float32)


def _col(b_ref):
    return b_ref[...].reshape(H, 1)


def _conv_layer(h, p, refs, ve):
    (W1e, b1e, W2e, b2e, Wq1, bq1, Wq2, Wn1, bn1, Wn2, bn2) = refs
    hn = _roll_l(h)
    pn = _roll_l(p)
    rel = pn - p
    dist = jnp.sqrt(jnp.sum(rel * rel, axis=0, keepdims=True))
    z = (_dot_t(W1e[0:H], h) + _dot_t(W1e[H:2 * H], hn)
         + _dot_t(W1e[2 * H:2 * H + 1], dist) + _col(b1e))
    eh = _silu(z)
    ea = _dot_t(W2e[...], eh) + _col(b2e)
    ph = _silu(_dot_t(Wq1[...], ea) + _col(bq1))
    dp = _dot_t(Wq2[...], ph)
    ea_m = ea * ve
    dp_m = dp * ve
    nu = ea_m + _roll_r(ea_m)
    pu = dp_m - _roll_r(dp_m)
    nh = _silu(_dot_t(Wn1[0:H], h) + _dot_t(Wn1[H:2 * H], nu) + _col(bn1))
    h2 = _dot_t(Wn2[...], nh) + _col(bn2)
    p2 = p + 0.1 * pu
    return h2, p2


def _tile_kernel(*args, T, N, G):
    (ap_ref, am_ref, ap_any, am_any), rest = args[:4], args[4:]
    wr = rest[:108]
    po_ref, mo_ref = rest[108:110]
    lo_ap, hi_ap, lo_am, hi_am, sems = rest[110:]

    W = T + 2 * HALO
    t = pl.program_id(0)

    @pl.when(t > 0)
    def _():
        pltpu.make_async_copy(
            ap_any.at[pl.ds(t * T - HALO, HALO)], lo_ap, sems.at[0]).start()
        pltpu.make_async_copy(
            am_any.at[pl.ds(t * T - HALO, HALO)], lo_am, sems.at[1]).start()

    @pl.when(t < G - 1)
    def _():
        pltpu.make_async_copy(
            ap_any.at[pl.ds((t + 1) * T, HALO)], hi_ap, sems.at[2]).start()
        pltpu.make_async_copy(
            am_any.at[pl.ds((t + 1) * T, HALO)], hi_am, sems.at[3]).start()

    @pl.when(t == 0)
    def _():
        lo_ap[...] = jnp.zeros((HALO, P_DIM), jnp.float32)
        lo_am[...] = jnp.zeros((HALO, A_DIM), jnp.float32)

    @pl.when(t == G - 1)
    def _():
        hi_ap[...] = jnp.zeros((HALO, P_DIM), jnp.float32)
        hi_am[...] = jnp.zeros((HALO, A_DIM), jnp.float32)

    @pl.when(t > 0)
    def _():
        pltpu.make_async_copy(
            ap_any.at[pl.ds(t * T - HALO, HALO)], lo_ap, sems.at[0]).wait()
        pltpu.make_async_copy(
            am_any.at[pl.ds(t * T - HALO, HALO)], lo_am, sems.at[1]).wait()

    @pl.when(t < G - 1)
    def _():
        pltpu.make_async_copy(
            ap_any.at[pl.ds((t + 1) * T, HALO)], hi_ap, sems.at[2]).wait()
        pltpu.make_async_copy(
            am_any.at[pl.ds((t + 1) * T, HALO)], hi_am, sems.at[3]).wait()

    apw = jnp.concatenate([lo_ap[...], ap_ref[...], hi_ap[...]], axis=0)
    amw = jnp.concatenate([lo_am[...], am_ref[...], hi_am[...]], axis=0)

    ia = jax.lax.broadcasted_iota(jnp.int32, (A_DIM, P_DIM), 0)
    il = jax.lax.broadcasted_iota(jnp.int32, (A_DIM, P_DIM), 1)
    R = (il // 3 == ia).astype(jnp.float32)
    jl = jax.lax.broadcasted_iota(jnp.int32, (P_DIM, 3), 0)
    jk = jax.lax.broadcasted_iota(jnp.int32, (P_DIM, 3), 1)
    S = (jl % 3 == jk).astype(jnp.float32)

    (We, be, Wp1, bp1, Wp2, bp2) = wr[:6]
    mask_rep = jnp.dot(amw, R, preferred_element_type=jnp.float32)
    wp = apw * mask_rep
    mp = jnp.dot(wp, S, preferred_element_type=jnp.float32)
    msum = jnp.sum(amw, axis=1, keepdims=True)
    mean_pos = mp / (msum + 1e-8)
    h0 = (jnp.dot(amw, We[...], preferred_element_type=jnp.float32) + be[...]
          + jnp.dot(_silu(jnp.dot(mean_pos, Wp1[...],
                                  preferred_element_type=jnp.float32)
                          + bp1[...]),
                    Wp2[...], preferred_element_type=jnp.float32)
          + bp2[...])

    hT = h0.T
    posT = mean_pos.T

    ids = jax.lax.broadcasted_iota(jnp.int32, (1, W), 1)
    g = ids + (t * T - HALO)
    ve = ((g >= 0) & (g < N - 1)).astype(jnp.float32)

    conv = wr[6:6 + 88]
    for i in range(4):
        hT, posT = _conv_layer(hT, posT, conv[11 * i:11 * i + 11], ve)

    (Wt1, bt1, Wt2, bt2, Wf1, bf1, Wf2, bf2) = wr[94:102]
    zt = _silu(_dot_t(Wt1[...], hT) + _col(bt1))
    zl = _dot_t(Wt2[...], zt) + _col(bt2)
    zf = _silu(_dot_t(Wf1[...], zl) + _col(bf1))
    hT = _dot_t(Wf2[...], zf) + _col(bf2)

    for i in range(4, 8):
        hT, posT = _conv_layer(hT, posT, conv[11 * i:11 * i + 11], ve)

    hF = hT[:, HALO:HALO + T].T

    (Wd1, bd1, Wd2, bd2, Wm, bm) = wr[102:108]
    hid = _silu(jnp.dot(hF, Wd1[...], preferred_element_type=jnp.float32)
                + bd1[...])
    po_ref[...] = (jnp.dot(hid, Wd2[...], preferred_element_type=jnp.float32)
                   + bd2[...])
    mo_ref[...] = (jnp.dot(hF, Wm[...], preferred_element_type=jnp.float32)
                   + bm[...])


def kernel(atom_positions, atom_mask, params):
    Bq, Lq, A = atom_mask.shape
    N = Bq * Lq
    T = 8192 if N % 8192 == 0 else N
    G = N // T

    ap = atom_positions.reshape(N, P_DIM)
    am = atom_mask.reshape(N, A_DIM)

    We, be = params["node_emb"]
    (Wp1, bp1), (Wp2, bp2) = params["pos_emb"]
    weights = [We, be[None, :], Wp1, bp1[None, :], Wp2, bp2[None, :]]
    for lp in params["enc"] + params["dec"]:
        (W1e, b1e), (W2e, b2e) = lp["edge"]
        (Wq1, bq1), Wq2 = lp["posm"]
        (Wn1, bn1), (Wn2, bn2) = lp["node"]
        weights += [W1e, b1e[None, :], W2e, b2e[None, :],
                    Wq1, bq1[None, :], Wq2,
                    Wn1, bn1[None, :], Wn2, bn2[None, :]]
    (Wt1, bt1), (Wt2, bt2) = params["to_latent"]
    (Wf1, bf1), (Wf2, bf2) = params["from_latent"]
    weights += [Wt1, bt1[None, :], Wt2, bt2[None, :],
                Wf1, bf1[None, :], Wf2, bf2[None, :]]
    (Wd1, bd1), (Wd2, bd2) = params["pos_dec"]
    Wm, bm = params["mask_dec"]
    weights += [Wd1, bd1[None, :], Wd2, bd2[None, :], Wm, bm[None, :]]

    def full(shape):
        nd = len(shape)
        return pl.BlockSpec(shape, lambda t, _n=nd: (0,) * _n)

    in_specs = [
        pl.BlockSpec((T, P_DIM), lambda t: (t, 0)),
        pl.BlockSpec((T, A_DIM), lambda t: (t, 0)),
        pl.BlockSpec(memory_space=pltpu.MemorySpace.HBM),
        pl.BlockSpec(memory_space=pltpu.MemorySpace.HBM),
    ] + [full(w.shape) for w in weights]
    out_specs = [
        pl.BlockSpec((T, P_DIM), lambda t: (t, 0)),
        pl.BlockSpec((T, A_DIM), lambda t: (t, 0)),
    ]
    out_shape = [
        jax.ShapeDtypeStruct((N, P_DIM), jnp.float32),
        jax.ShapeDtypeStruct((N, A_DIM), jnp.float32),
    ]
    scratch_shapes = [
        pltpu.VMEM((HALO, P_DIM), jnp.float32),
        pltpu.VMEM((HALO, P_DIM), jnp.float32),
        pltpu.VMEM((HALO, A_DIM), jnp.float32),
        pltpu.VMEM((HALO, A_DIM), jnp.float32),
        pltpu.SemaphoreType.DMA((4,)),
    ]

    po, mo = pl.pallas_call(
        functools.partial(_tile_kernel, T=T, N=N, G=G),
        grid=(G,),
        in_specs=in_specs,
        out_specs=out_specs,
        out_shape=out_shape,
        scratch_shapes=scratch_shapes,
    )(ap, am, ap, am, *weights)

    return (po.reshape(Bq, Lq, A, 3), mo.reshape(Bq, Lq, A))
